# Initial kernel scaffold; baseline (speedup 1.0000x reference)
import jax
import jax.numpy as jnp
from jax.experimental import pallas as pl


def kernel(x0, L_idx, L_val, Lu_idx, Lu_val, Ld_idx, Ld_val, W_p, W_s, W_lu, W_ld):
    raise NotImplementedError("write your pallas kernel here")


# speedup vs baseline: 4.9626x; 4.9626x over previous
import functools

import jax
import jax.numpy as jnp
from jax import lax
from jax.experimental import pallas as pl
from jax.experimental.pallas import tpu as pltpu
from jax.experimental.pallas import tpu_sc as plsc

N = 10000
NNZ = 320000
D = 128
NC = 2
NS = 16
LANES = 16
EPT = NNZ // NS
K = 80
NCHUNK = EPT // K
SC_CH = 10
ZTILES = 10
RPT = N // ZTILES
MM_BLK = 1000


def _mm_kernel(x_ref, w_ref, y_ref):
    y_ref[...] = lax.dot_general(
        x_ref[...], w_ref[0],
        dimension_numbers=(((1,), (0,)), ((), ())),
        preferred_element_type=jnp.float32,
    )


def _matmul_stacked(x0, w2):
    grid = (2, N // MM_BLK)
    return pl.pallas_call(
        _mm_kernel,
        grid=grid,
        in_specs=[
            pl.BlockSpec((MM_BLK, D), lambda w, i: (i, 0)),
            pl.BlockSpec((1, D, D), lambda w, i: (w, 0, 0)),
        ],
        out_specs=pl.BlockSpec((MM_BLK, D),
                               lambda w, i: (w * (N // MM_BLK) + i, 0)),
        out_shape=jax.ShapeDtypeStruct((2 * N, D), jnp.float32),
    )(x0, w2)


def _epi_kernel(x_ref, ws_ref, sp_ref, sl_ref, o_ref):
    hs = lax.dot_general(
        x_ref[...], ws_ref[...],
        dimension_numbers=(((1,), (0,)), ((), ())),
        preferred_element_type=jnp.float32,
    )
    o_ref[...] = jnp.tanh(hs) + jnp.tanh(sp_ref[...]) + jnp.tanh(sl_ref[...])


def _epilogue(x0, w_s, s_all):
    grid = (N // MM_BLK,)
    return pl.pallas_call(
        _epi_kernel,
        grid=grid,
        in_specs=[
            pl.BlockSpec((MM_BLK, D), lambda i: (i, 0)),
            pl.BlockSpec((D, D), lambda i: (0, 0)),
            pl.BlockSpec((MM_BLK, D), lambda i: (i, 0)),
            pl.BlockSpec((MM_BLK, D), lambda i: (i + N // MM_BLK, 0)),
        ],
        out_specs=pl.BlockSpec((MM_BLK, D), lambda i: (i, 0)),
        out_shape=jax.ShapeDtypeStruct((N, D), jnp.float32),
    )(x0, w_s, s_all, s_all)


_SC_MESH = plsc.VectorSubcoreMesh(core_axis_name="c", subcore_axis_name="s")


@functools.partial(
    pl.kernel,
    out_type=jax.ShapeDtypeStruct((2 * N, D), jnp.float32),
    mesh=_SC_MESH,
    scratch_types=[
        pltpu.VMEM_SHARED((N, D), jnp.float32),
        pltpu.VMEM((SC_CH, K), jnp.int32),
        pltpu.VMEM((SC_CH, K), jnp.int32),
        pltpu.VMEM((SC_CH * K,), jnp.float32),
        pltpu.VMEM((K, D), jnp.float32),
        pltpu.SemaphoreType.DMA,
    ],
    compiler_params=pltpu.CompilerParams(use_tc_tiling_on_sc=False),
)
def _spmm_sc(y_hbm, colsh, rowsh, valsh, zer_hbm, out_hbm,
             accum, colv, rowv, valv, gath, gsem):
    cid = lax.axis_index("c")
    sid = lax.axis_index("s")
    @pl.when(sid < ZTILES)
    def _zero():
        pltpu.sync_copy(zer_hbm, accum.at[pl.ds(sid * RPT, RPT)])
    plsc.subcore_barrier()

    def super_body(sc, carry):
        base = sc * SC_CH
        pltpu.sync_copy(colsh.at[cid, sid, pl.ds(base, SC_CH)], colv)
        pltpu.sync_copy(rowsh.at[cid, sid, pl.ds(base, SC_CH)], rowv)
        pltpu.sync_copy(valsh.at[cid, sid, pl.ds(base * K, SC_CH * K)], valv)

        def chunk_body(jj, c2):
            pltpu.async_copy(y_hbm.at[colv.at[jj]], gath, gsem).wait()

            def grp_body(g, c3):
                vv16 = valv[pl.ds(jj * K + g * LANES, LANES)]
                for l in range(LANES):
                    e = g * LANES + l
                    bc = lax.broadcast(vv16[l], (LANES,))
                    for c in range(D // LANES):
                        sl = pl.ds(c * LANES, LANES)
                        gath[e, sl] = gath[e, sl] * bc
                return c3

            lax.fori_loop(0, K // LANES, grp_body, 0)
            pltpu.sync_copy(gath, accum.at[rowv.at[jj]], add=True)
            return c2

        lax.fori_loop(0, SC_CH, chunk_body, 0)
        return carry

    lax.fori_loop(0, NCHUNK // SC_CH, super_body, 0)
    plsc.subcore_barrier()

    @pl.when(sid < ZTILES)
    def _copy_out():
        pltpu.sync_copy(accum.at[pl.ds(sid * RPT, RPT)],
                        out_hbm.at[pl.ds(cid * N + sid * RPT, RPT)])


def kernel(x0, L_idx, L_val, Lu_idx, Lu_val, Ld_idx, Ld_val,
           W_p, W_s, W_lu, W_ld):
    del Lu_idx, Lu_val, W_lu
    y_all = _matmul_stacked(x0, jnp.stack([W_p, W_ld]))
    cols = jnp.stack([L_idx[1], Ld_idx[1] + N]).reshape(NC, NS, NCHUNK, K)
    rows = jnp.stack([L_idx[0], Ld_idx[0]]).reshape(NC, NS, NCHUNK, K)
    vals = jnp.stack([L_val, Ld_val]).reshape(NC, NS, EPT)
    zer = jnp.zeros((RPT, D), jnp.float32)
    s_all = _spmm_sc(y_all, cols, rows, vals, zer)
    return _epilogue(x0, W_s, s_all)
